# baseline (speedup 1.0000x reference)
import jax
import jax.numpy as jnp
from jax.experimental import pallas as pl
from jax.experimental.pallas import tpu as pltpu

N, FEAT, HID, OUT = 4096, 256, 256, 128
BM = 512
M_BLOCKS = N // BM
BM1 = 512
M1_BLOCKS = N // BM1
STREAM_STEPS = 2 * M_BLOCKS


def _body(adj_ref, emb1_ref, w1_ref, b1_ref, w2_ref, b2_ref, out_ref,
          adj_scr, xw1_scr, xw2_scr):
    i = pl.program_id(0)

    @pl.when(i == 0)
    def _():
        w = w1_ref[...].astype(jnp.bfloat16)
        xw1_scr[...] = jnp.dot(
            emb1_ref[...], w, preferred_element_type=jnp.float32
        ).astype(jnp.bfloat16)

    @pl.when(i < M_BLOCKS)
    def _():
        adj_scr[pl.ds(i * BM, BM), :] = adj_ref[...].astype(jnp.bfloat16)

    @pl.when(jnp.logical_and(i >= M_BLOCKS, i < STREAM_STEPS))
    def _():
        m = i - M_BLOCKS
        a = adj_scr[pl.ds(m * BM, BM), :]
        acc = jnp.dot(a, xw1_scr[...], preferred_element_type=jnp.float32)
        x1 = jnp.maximum(acc + b1_ref[...], 0.0).astype(jnp.bfloat16)
        w2 = w2_ref[...].astype(jnp.bfloat16)
        xw2_scr[pl.ds(m * BM, BM), :] = jnp.dot(
            x1, w2, preferred_element_type=jnp.float32).astype(jnp.bfloat16)

    @pl.when(i >= STREAM_STEPS)
    def _():
        m1 = i - STREAM_STEPS
        a = adj_scr[pl.ds(m1 * BM1, BM1), :]
        out_ref[...] = jnp.dot(
            a, xw2_scr[...], preferred_element_type=jnp.float32) + b2_ref[...]


def kernel(adj_matrix, emb1, W1, b1, W2, b2):
    b1r = b1.reshape(1, HID)
    b2r = b2.reshape(1, OUT)
    emb1_bf = emb1.astype(jnp.bfloat16)
    return pl.pallas_call(
        _body,
        grid=(STREAM_STEPS + M1_BLOCKS,),
        in_specs=[
            pl.BlockSpec((BM, N), lambda i: (jnp.minimum(i, M_BLOCKS - 1), 0)),
            pl.BlockSpec((N, FEAT), lambda i: (0, 0)),
            pl.BlockSpec((FEAT, HID), lambda i: (0, 0)),
            pl.BlockSpec((1, HID), lambda i: (0, 0)),
            pl.BlockSpec((HID, OUT), lambda i: (0, 0)),
            pl.BlockSpec((1, OUT), lambda i: (0, 0)),
        ],
        out_specs=pl.BlockSpec(
            (BM1, OUT),
            lambda i: (jnp.maximum(i - STREAM_STEPS, 0), 0)),
        out_shape=jax.ShapeDtypeStruct((N, OUT), jnp.float32),
        scratch_shapes=[
            pltpu.VMEM((N, N), jnp.bfloat16),
            pltpu.VMEM((N, FEAT), jnp.bfloat16),
            pltpu.VMEM((N, OUT), jnp.bfloat16),
        ],
    )(adj_matrix, emb1_bf, W1, b1r, W2, b2r)


# speedup vs baseline: 1.1478x; 1.1478x over previous
import jax
import jax.numpy as jnp
from jax.experimental import pallas as pl
from jax.experimental.pallas import tpu as pltpu

N, FEAT, HID, OUT = 4096, 256, 256, 128
BM = 512
M_BLOCKS = N // BM
HALF = M_BLOCKS // 2
L1_END = M_BLOCKS + 1


def _layer1(src, slot, xw1_scr, xw2_scr, b1_ref, w2_ref, blk):
    a = src[pl.ds(slot * BM, BM), :]
    acc = jnp.dot(a, xw1_scr[...], preferred_element_type=jnp.float32)
    x1 = jnp.maximum(acc + b1_ref[...], 0.0).astype(jnp.bfloat16)
    w2 = w2_ref[...].astype(jnp.bfloat16)
    xw2_scr[pl.ds(blk * BM, BM), :] = jnp.dot(
        x1, w2, preferred_element_type=jnp.float32).astype(jnp.bfloat16)


def _body(adj_ref, emb1_ref, w1_ref, b1_ref, w2_ref, b2_ref, out_ref,
          adj_a, adj_b, xw1_scr, xw2_scr):
    i = pl.program_id(0)
    par = jax.lax.rem(i, 2)

    @pl.when(i == 0)
    def _():
        w = w1_ref[...].astype(jnp.bfloat16)
        xw1_scr[...] = jnp.dot(
            emb1_ref[...], w, preferred_element_type=jnp.float32
        ).astype(jnp.bfloat16)
        adj_a[pl.ds(0, BM), :] = adj_ref[...].astype(jnp.bfloat16)

    @pl.when(jnp.logical_and(par == 0, jnp.logical_and(i >= 2, i < M_BLOCKS)))
    def _():
        adj_a[pl.ds((i // 2) * BM, BM), :] = adj_ref[...].astype(jnp.bfloat16)
        _layer1(adj_b, (i - 1) // 2, xw1_scr, xw2_scr, b1_ref, w2_ref, i - 1)

    @pl.when(jnp.logical_and(par == 1, i < M_BLOCKS))
    def _():
        adj_b[pl.ds((i // 2) * BM, BM), :] = adj_ref[...].astype(jnp.bfloat16)
        _layer1(adj_a, (i - 1) // 2, xw1_scr, xw2_scr, b1_ref, w2_ref, i - 1)

    @pl.when(i == M_BLOCKS)
    def _():
        _layer1(adj_b, (M_BLOCKS - 1) // 2, xw1_scr, xw2_scr, b1_ref, w2_ref,
                M_BLOCKS - 1)

    m1 = i - L1_END

    @pl.when(jnp.logical_and(i > M_BLOCKS, jax.lax.rem(m1, 2) == 0))
    def _():
        a = adj_a[pl.ds((m1 // 2) * BM, BM), :]
        out_ref[...] = jnp.dot(
            a, xw2_scr[...], preferred_element_type=jnp.float32) + b2_ref[...]

    @pl.when(jnp.logical_and(i > M_BLOCKS, jax.lax.rem(m1, 2) == 1))
    def _():
        a = adj_b[pl.ds((m1 // 2) * BM, BM), :]
        out_ref[...] = jnp.dot(
            a, xw2_scr[...], preferred_element_type=jnp.float32) + b2_ref[...]


def kernel(adj_matrix, emb1, W1, b1, W2, b2):
    b1r = b1.reshape(1, HID)
    b2r = b2.reshape(1, OUT)
    emb1_bf = emb1.astype(jnp.bfloat16)
    return pl.pallas_call(
        _body,
        grid=(L1_END + M_BLOCKS,),
        in_specs=[
            pl.BlockSpec((BM, N), lambda i: (jnp.minimum(i, M_BLOCKS - 1), 0)),
            pl.BlockSpec((N, FEAT), lambda i: (0, 0)),
            pl.BlockSpec((FEAT, HID), lambda i: (0, 0)),
            pl.BlockSpec((1, HID), lambda i: (0, 0)),
            pl.BlockSpec((HID, OUT), lambda i: (0, 0)),
            pl.BlockSpec((1, OUT), lambda i: (0, 0)),
        ],
        out_specs=pl.BlockSpec(
            (BM, OUT),
            lambda i: (jnp.maximum(i - L1_END, 0), 0)),
        out_shape=jax.ShapeDtypeStruct((N, OUT), jnp.float32),
        scratch_shapes=[
            pltpu.VMEM((HALF * BM, N), jnp.bfloat16),
            pltpu.VMEM((HALF * BM, N), jnp.bfloat16),
            pltpu.VMEM((N, FEAT), jnp.bfloat16),
            pltpu.VMEM((N, OUT), jnp.bfloat16),
        ],
    )(adj_matrix, emb1_bf, W1, b1r, W2, b2r)


# speedup vs baseline: 1.2247x; 1.0670x over previous
import jax
import jax.numpy as jnp
from jax.experimental import pallas as pl
from jax.experimental.pallas import tpu as pltpu

N, FEAT, HID, OUT = 4096, 256, 256, 128
BM = 512
M_BLOCKS = N // BM


def _body(adj_ref, emb1_ref, w1_ref, b1_ref, w2_ref, b2_ref, out_ref,
          adj_scr, xw1_scr, xw2_scr):
    i = pl.program_id(0)
    m = jax.lax.rem(i, M_BLOCKS)

    @pl.when(i == 0)
    def _():
        w = w1_ref[...].astype(jnp.bfloat16)
        xw1_scr[...] = jnp.dot(
            emb1_ref[...], w, preferred_element_type=jnp.float32
        ).astype(jnp.bfloat16)

    @pl.when(i < M_BLOCKS)
    def _():
        a = adj_ref[...].astype(jnp.bfloat16)
        adj_scr[pl.ds(m * BM, BM), :] = a
        acc = jnp.dot(a, xw1_scr[...], preferred_element_type=jnp.float32)
        x1 = jnp.maximum(acc + b1_ref[...], 0.0).astype(jnp.bfloat16)
        w2 = w2_ref[...].astype(jnp.bfloat16)
        xw2_scr[pl.ds(m * BM, BM), :] = jnp.dot(
            x1, w2, preferred_element_type=jnp.float32).astype(jnp.bfloat16)

    @pl.when(i >= M_BLOCKS)
    def _():
        a = adj_scr[pl.ds(m * BM, BM), :]
        out_ref[...] = jnp.dot(
            a, xw2_scr[...], preferred_element_type=jnp.float32) + b2_ref[...]


def kernel(adj_matrix, emb1, W1, b1, W2, b2):
    b1r = b1.reshape(1, HID)
    b2r = b2.reshape(1, OUT)
    emb1_bf = emb1.astype(jnp.bfloat16)
    return pl.pallas_call(
        _body,
        grid=(2 * M_BLOCKS,),
        in_specs=[
            pl.BlockSpec((BM, N), lambda i: (jnp.minimum(i, M_BLOCKS - 1), 0)),
            pl.BlockSpec((N, FEAT), lambda i: (0, 0)),
            pl.BlockSpec((FEAT, HID), lambda i: (0, 0)),
            pl.BlockSpec((1, HID), lambda i: (0, 0)),
            pl.BlockSpec((HID, OUT), lambda i: (0, 0)),
            pl.BlockSpec((1, OUT), lambda i: (0, 0)),
        ],
        out_specs=pl.BlockSpec((BM, OUT), lambda i: (jax.lax.rem(i, M_BLOCKS), 0)),
        out_shape=jax.ShapeDtypeStruct((N, OUT), jnp.float32),
        scratch_shapes=[
            pltpu.VMEM((N, N), jnp.bfloat16),
            pltpu.VMEM((N, FEAT), jnp.bfloat16),
            pltpu.VMEM((N, OUT), jnp.bfloat16),
        ],
    )(adj_matrix, emb1_bf, W1, b1r, W2, b2r)


# speedup vs baseline: 1.3282x; 1.0845x over previous
import jax
import jax.numpy as jnp
from jax.experimental import pallas as pl
from jax.experimental.pallas import tpu as pltpu

N, FEAT, HID, OUT = 4096, 256, 256, 128
BM = 512
M_BLOCKS = N // BM


def _body(adj_ref, emb1_ref, w1_ref, b1_ref, w2_ref, b2_ref, out_ref,
          adj_scr, xw1_scr, xw2_scr):
    i = pl.program_id(0)
    m = jax.lax.rem(i, M_BLOCKS)

    @pl.when(i == 0)
    def _():
        e = emb1_ref[...].astype(jnp.bfloat16)
        w = w1_ref[...].astype(jnp.bfloat16)
        xw1_scr[...] = jnp.dot(
            e, w, preferred_element_type=jnp.float32).astype(jnp.bfloat16)

    @pl.when(i < M_BLOCKS)
    def _():
        a = adj_ref[...].astype(jnp.bfloat16)
        adj_scr[pl.ds(m * BM, BM), :] = a
        acc = jnp.dot(a, xw1_scr[...], preferred_element_type=jnp.float32)
        x1 = jnp.maximum(acc + b1_ref[...], 0.0).astype(jnp.bfloat16)
        w2 = w2_ref[...].astype(jnp.bfloat16)
        xw2_scr[pl.ds(m * BM, BM), :] = jnp.dot(
            x1, w2, preferred_element_type=jnp.float32).astype(jnp.bfloat16)

    @pl.when(i >= M_BLOCKS)
    def _():
        a = adj_scr[pl.ds(m * BM, BM), :]
        out_ref[...] = jnp.dot(
            a, xw2_scr[...], preferred_element_type=jnp.float32) + b2_ref[...]


def kernel(adj_matrix, emb1, W1, b1, W2, b2):
    b1r = b1.reshape(1, HID)
    b2r = b2.reshape(1, OUT)
    return pl.pallas_call(
        _body,
        grid=(2 * M_BLOCKS,),
        in_specs=[
            pl.BlockSpec((BM, N), lambda i: (jnp.minimum(i, M_BLOCKS - 1), 0)),
            pl.BlockSpec((N, FEAT), lambda i: (0, 0)),
            pl.BlockSpec((FEAT, HID), lambda i: (0, 0)),
            pl.BlockSpec((1, HID), lambda i: (0, 0)),
            pl.BlockSpec((HID, OUT), lambda i: (0, 0)),
            pl.BlockSpec((1, OUT), lambda i: (0, 0)),
        ],
        out_specs=pl.BlockSpec((BM, OUT), lambda i: (jax.lax.rem(i, M_BLOCKS), 0)),
        out_shape=jax.ShapeDtypeStruct((N, OUT), jnp.float32),
        scratch_shapes=[
            pltpu.VMEM((N, N), jnp.bfloat16),
            pltpu.VMEM((N, FEAT), jnp.bfloat16),
            pltpu.VMEM((N, OUT), jnp.bfloat16),
        ],
    )(adj_matrix, emb1, W1, b1r, W2, b2r)
